# baseline (speedup 1.0000x reference)
import functools

import jax
import jax.numpy as jnp
from jax import lax
from jax.experimental import pallas as pl
from jax.experimental.pallas import tpu as pltpu
from jax.experimental.pallas import tpu_sc as plsc

NC = 2
NS = 16
NW = NC * NS
LANES = 16
CHUNK = 64


def _sc_segment_sum(feat, src2, dst2, w2, zeros, n_nodes):
    n_chunks, _ = src2.shape
    d = feat.shape[1]
    cpt = n_chunks // NW
    rpt = n_nodes // NS
    mesh = plsc.VectorSubcoreMesh(core_axis_name="c", subcore_axis_name="s")

    hcpt = cpt // 4

    @functools.partial(
        pl.kernel,
        out_type=jax.ShapeDtypeStruct((NC, n_nodes, d), jnp.float32),
        mesh=mesh,
        scratch_types=[
            pltpu.VMEM((hcpt, CHUNK), jnp.int32),
            pltpu.VMEM((hcpt, CHUNK), jnp.int32),
            pltpu.VMEM((hcpt, CHUNK), jnp.float32),
            pltpu.VMEM((CHUNK, 128), jnp.float32),
            pltpu.VMEM((CHUNK, 128), jnp.float32),
            pltpu.VMEM((CHUNK, 128), jnp.float32),
            pltpu.VMEM((CHUNK, 128), jnp.float32),
            pltpu.VMEM_SHARED((n_nodes, 128), jnp.float32),
            pltpu.SemaphoreType.DMA,
            pltpu.SemaphoreType.DMA,
            pltpu.SemaphoreType.DMA,
            pltpu.SemaphoreType.DMA,
        ],
    )
    def run(feat_hbm, src_hbm, dst_hbm, w_hbm, zeros_hbm, out_hbm,
            sidx, didx, wv, g0, g1, s0, s1, acc, gs0, gs1, ss0, ss1):
        cid = lax.axis_index("c")
        tid = lax.axis_index("s")
        wid = cid * NS + tid

        r0 = tid * rpt
        pltpu.sync_copy(zeros_hbm.at[pl.ds(r0, rpt)], acc.at[pl.ds(r0, rpt)])
        plsc.subcore_barrier()

        bufs = ((g0, gs0, s0, ss0), (g1, gs1, s1, ss1))

        def scale(g, s, i):
            def group_body(gi, c2):
                w16 = wv[i, pl.ds(gi * LANES, LANES)]

                def col_body(k, c3):
                    sl = pl.ds(k * LANES, LANES)
                    for j in range(LANES):
                        ws = w16[j]
                        e = gi * LANES + j
                        s[e, sl] = g[e, sl] * ws
                    return c3

                lax.fori_loop(0, d // LANES, col_body, 0)
                return c2

            lax.fori_loop(0, CHUNK // LANES, group_body, 0)

        def stage_body(h, carry0):
            c0 = wid * cpt + h * hcpt
            pltpu.sync_copy(src_hbm.at[pl.ds(c0, hcpt)], sidx)
            pltpu.sync_copy(dst_hbm.at[pl.ds(c0, hcpt)], didx)
            pltpu.sync_copy(w_hbm.at[pl.ds(c0, hcpt)], wv)

            pltpu.async_copy(feat_hbm.at[sidx.at[0]], g0, gs0)
            pltpu.async_copy(feat_hbm.at[sidx.at[1]], g1, gs1)

            for b, (g, gsem, s, ssem) in enumerate(bufs):
                pltpu.make_async_copy(feat_hbm.at[sidx.at[b]], g, gsem).wait()
                scale(g, s, b)
                pltpu.async_copy(feat_hbm.at[sidx.at[b + 2]], g, gsem)
                pltpu.async_copy(s, acc.at[didx.at[b]], ssem, add=True)

            def outer_body(io, carry):
                for b, (g, gsem, s, ssem) in enumerate(bufs):
                    i = 2 * io + b
                    pltpu.make_async_copy(
                        feat_hbm.at[sidx.at[i]], g, gsem).wait()
                    pltpu.make_async_copy(
                        s, acc.at[didx.at[i - 2]], ssem).wait()
                    scale(g, s, i)

                    @pl.when(i + 2 < hcpt)
                    def _(g=g, gsem=gsem, i=i):
                        pltpu.async_copy(feat_hbm.at[sidx.at[i + 2]], g, gsem)

                    pltpu.async_copy(s, acc.at[didx.at[i]], ssem, add=True)

                return carry

            lax.fori_loop(1, hcpt // 2, outer_body, 0)

            pltpu.make_async_copy(s0, acc.at[didx.at[hcpt - 2]], ss0).wait()
            pltpu.make_async_copy(s1, acc.at[didx.at[hcpt - 1]], ss1).wait()
            return carry0

        lax.fori_loop(0, 4, stage_body, 0)

        plsc.subcore_barrier()
        pltpu.sync_copy(acc.at[pl.ds(r0, rpt)],
                        out_hbm.at[cid, pl.ds(r0, rpt)])

    return run(feat, src2, dst2, w2, zeros)


def _tc_linear(partials, w, b, n):
    d = partials.shape[2]
    o = w.shape[1]
    br = 1000

    def body(p_ref, w_ref, b_ref, o_ref):
        h = p_ref[0] + p_ref[1]
        o_ref[...] = (
            jnp.dot(h, w_ref[...], preferred_element_type=jnp.float32)
            + b_ref[...]
        )

    return pl.pallas_call(
        body,
        grid=(n // br,),
        in_specs=[
            pl.BlockSpec((2, br, d), lambda i: (0, i, 0)),
            pl.BlockSpec((d, o), lambda i: (0, 0)),
            pl.BlockSpec((1, o), lambda i: (0, 0)),
        ],
        out_specs=pl.BlockSpec((br, o), lambda i: (i, 0)),
        out_shape=jax.ShapeDtypeStruct((n, o), jnp.float32),
    )(partials, w, b.reshape(1, o))


def kernel(feat, edge_index, edge_weight, W, b):
    n_nodes, d = feat.shape
    src = edge_index[0].astype(jnp.int32)
    dst = edge_index[1].astype(jnp.int32)
    w = edge_weight.astype(jnp.float32)

    n_edges = src.shape[0]
    group = NW * CHUNK * 8
    ep = -(-n_edges // group) * group
    pad = ep - n_edges
    if pad:
        fill = jnp.arange(pad, dtype=jnp.int32) % n_nodes
        src = jnp.concatenate([src, fill])
        dst = jnp.concatenate([dst, fill])
        w = jnp.pad(w, (0, pad))
    src2 = src.reshape(ep // CHUNK, CHUNK)
    dst2 = dst.reshape(ep // CHUNK, CHUNK)
    w2 = w.reshape(ep // CHUNK, CHUNK)

    np_pad = -(-n_nodes // (NS * 8)) * (NS * 8)
    zeros = jnp.zeros((np_pad, d), jnp.float32)
    partials = _sc_segment_sum(feat, src2, dst2, w2, zeros, np_pad)
    return _tc_linear(partials, W, b, n_nodes)


# speedup vs baseline: 2.3693x; 2.3693x over previous
import functools

import jax
import jax.numpy as jnp
from jax import lax
from jax.experimental import pallas as pl
from jax.experimental.pallas import tpu as pltpu
from jax.experimental.pallas import tpu_sc as plsc

NC = 2
NS = 16
NW = NC * NS
LANES = 16
CHUNK = 64


def _sc_segment_sum(feat, src2, dst2, w2, zeros, n_nodes):
    n_chunks, _ = src2.shape
    d = feat.shape[1]
    cpt = n_chunks // NW
    rpt = n_nodes // NS
    mesh = plsc.VectorSubcoreMesh(core_axis_name="c", subcore_axis_name="s")

    hcpt = cpt // 2

    @functools.partial(
        pl.kernel,
        out_type=jax.ShapeDtypeStruct((NC, n_nodes, d), jnp.float32),
        mesh=mesh,
        scratch_types=[
            pltpu.VMEM((hcpt, CHUNK), jnp.int32),
            pltpu.VMEM((hcpt, CHUNK), jnp.int32),
            pltpu.VMEM((hcpt, CHUNK), jnp.float32),
            pltpu.VMEM((CHUNK, 128), jnp.float32),
            pltpu.VMEM((CHUNK, 128), jnp.float32),
            pltpu.VMEM_SHARED((n_nodes, 128), jnp.float32),
            pltpu.SemaphoreType.DMA,
            pltpu.SemaphoreType.DMA,
        ],
    )
    def run(feat_hbm, src_hbm, dst_hbm, w_hbm, zeros_hbm, out_hbm,
            sidx, didx, wv, rows0, rows1, acc, sem0, sem1):
        cid = lax.axis_index("c")
        tid = lax.axis_index("s")
        wid = cid * NS + tid

        r0 = tid * rpt
        pltpu.sync_copy(zeros_hbm.at[pl.ds(r0, rpt)], acc.at[pl.ds(r0, rpt)])
        plsc.subcore_barrier()

        bufs = ((rows0, sem0), (rows1, sem1))

        def stage_body(h, carry0):
            c0 = wid * cpt + h * hcpt
            pltpu.sync_copy(src_hbm.at[pl.ds(c0, hcpt)], sidx)
            pltpu.sync_copy(dst_hbm.at[pl.ds(c0, hcpt)], didx)
            pltpu.sync_copy(w_hbm.at[pl.ds(c0, hcpt)], wv)

            pltpu.async_copy(feat_hbm.at[sidx.at[0]], rows0, sem0)
            pltpu.async_copy(feat_hbm.at[sidx.at[1]], rows1, sem1)

            def outer_body(io, carry):
                for b, (rows, sem) in enumerate(bufs):
                    i = 2 * io + b
                    pltpu.make_async_copy(
                        feat_hbm.at[sidx.at[i]], rows, sem).wait()

                    def group_body(g, c2, rows=rows, i=i):
                        w16 = wv[i, pl.ds(g * LANES, LANES)]
                        for j in range(LANES):
                            ws = w16[j]
                            e = g * LANES + j
                            for k in range(d // LANES):
                                sl = pl.ds(k * LANES, LANES)
                                rows[e, sl] = rows[e, sl] * ws
                        return c2

                    lax.fori_loop(0, CHUNK // LANES, group_body, 0)
                    pltpu.sync_copy(rows, acc.at[didx.at[i]], add=True)

                    @pl.when(i + 2 < hcpt)
                    def _(rows=rows, sem=sem, i=i):
                        pltpu.async_copy(feat_hbm.at[sidx.at[i + 2]], rows, sem)

                return carry

            lax.fori_loop(0, hcpt // 2, outer_body, 0)
            return carry0

        lax.fori_loop(0, 2, stage_body, 0)

        plsc.subcore_barrier()
        pltpu.sync_copy(acc.at[pl.ds(r0, rpt)],
                        out_hbm.at[cid, pl.ds(r0, rpt)])

    return run(feat, src2, dst2, w2, zeros)


def _tc_linear(partials, w, b, n):
    d = partials.shape[2]
    o = w.shape[1]
    br = 1000

    def body(p_ref, w_ref, b_ref, o_ref):
        h = p_ref[0] + p_ref[1]
        o_ref[...] = (
            jnp.dot(h, w_ref[...], preferred_element_type=jnp.float32)
            + b_ref[...]
        )

    return pl.pallas_call(
        body,
        grid=(n // br,),
        in_specs=[
            pl.BlockSpec((2, br, d), lambda i: (0, i, 0)),
            pl.BlockSpec((d, o), lambda i: (0, 0)),
            pl.BlockSpec((1, o), lambda i: (0, 0)),
        ],
        out_specs=pl.BlockSpec((br, o), lambda i: (i, 0)),
        out_shape=jax.ShapeDtypeStruct((n, o), jnp.float32),
    )(partials, w, b.reshape(1, o))


def kernel(feat, edge_index, edge_weight, W, b):
    n_nodes, d = feat.shape
    src = edge_index[0].astype(jnp.int32)
    dst = edge_index[1].astype(jnp.int32)
    w = edge_weight.astype(jnp.float32)

    n_edges = src.shape[0]
    group = NW * CHUNK * 8
    ep = -(-n_edges // group) * group
    pad = ep - n_edges
    if pad:
        fill = jnp.arange(pad, dtype=jnp.int32) % n_nodes
        src = jnp.concatenate([src, fill])
        dst = jnp.concatenate([dst, fill])
        w = jnp.pad(w, (0, pad))
    src2 = src.reshape(ep // CHUNK, CHUNK)
    dst2 = dst.reshape(ep // CHUNK, CHUNK)
    w2 = w.reshape(ep // CHUNK, CHUNK)

    np_pad = -(-n_nodes // (NS * 8)) * (NS * 8)
    zeros = jnp.zeros((np_pad, d), jnp.float32)
    partials = _sc_segment_sum(feat, src2, dst2, w2, zeros, np_pad)
    return _tc_linear(partials, W, b, n_nodes)


# speedup vs baseline: 2.5357x; 1.0703x over previous
import functools

import jax
import jax.numpy as jnp
from jax import lax
from jax.experimental import pallas as pl
from jax.experimental.pallas import tpu as pltpu
from jax.experimental.pallas import tpu_sc as plsc

NC = 2
NS = 16
NW = NC * NS
LANES = 16
CHUNK = 64


def _sc_segment_sum(feat, src2, dst2, w2, zeros, n_nodes):
    n_chunks, _ = src2.shape
    d = feat.shape[1]
    cpt = n_chunks // NW
    rpt = n_nodes // NS
    mesh = plsc.VectorSubcoreMesh(core_axis_name="c", subcore_axis_name="s")

    hcpt = cpt // 4

    @functools.partial(
        pl.kernel,
        out_type=jax.ShapeDtypeStruct((NC, n_nodes, d), jnp.float32),
        mesh=mesh,
        scratch_types=[
            pltpu.VMEM((hcpt, CHUNK), jnp.int32),
            pltpu.VMEM((hcpt, CHUNK), jnp.int32),
            pltpu.VMEM((hcpt, CHUNK), jnp.float32),
            pltpu.VMEM((CHUNK, 128), jnp.float32),
            pltpu.VMEM((CHUNK, 128), jnp.float32),
            pltpu.VMEM((CHUNK, 128), jnp.float32),
            pltpu.VMEM((CHUNK, 128), jnp.float32),
            pltpu.VMEM_SHARED((n_nodes, 128), jnp.float32),
            pltpu.SemaphoreType.DMA,
            pltpu.SemaphoreType.DMA,
            pltpu.SemaphoreType.DMA,
            pltpu.SemaphoreType.DMA,
            pltpu.SemaphoreType.DMA,
            pltpu.SemaphoreType.DMA,
            pltpu.SemaphoreType.DMA,
            pltpu.SemaphoreType.DMA,
        ],
    )
    def run(feat_hbm, src_hbm, dst_hbm, w_hbm, zeros_hbm, out_hbm,
            sidx, didx, wv, r0b, r1b, r2b, r3b, acc,
            gs0, gs1, gs2, gs3, ss0, ss1, ss2, ss3):
        cid = lax.axis_index("c")
        tid = lax.axis_index("s")
        wid = cid * NS + tid

        r0 = tid * rpt
        pltpu.sync_copy(zeros_hbm.at[pl.ds(r0, rpt)], acc.at[pl.ds(r0, rpt)])
        plsc.subcore_barrier()

        bufs = ((r0b, gs0, ss0), (r1b, gs1, ss1),
                (r2b, gs2, ss2), (r3b, gs3, ss3))

        def scale(rows, i):
            def group_body(gi, c2):
                w16 = wv[i, pl.ds(gi * LANES, LANES)]
                for j in range(LANES):
                    ws = w16[j]
                    e = gi * LANES + j
                    for k in range(d // LANES):
                        sl = pl.ds(k * LANES, LANES)
                        rows[e, sl] = rows[e, sl] * ws
                return c2

            lax.fori_loop(0, CHUNK // LANES, group_body, 0)

        def chunk_step(i, b, wait_sc, fire_g):
            rows, gsem, ssem = bufs[b]
            pltpu.make_async_copy(feat_hbm.at[sidx.at[i]], rows, gsem).wait()
            scale(rows, i)
            pltpu.async_copy(rows, acc.at[didx.at[i]], ssem, add=True)
            nb = (b + 2) % 4
            nrows, ngsem, nssem = bufs[nb]
            if wait_sc:
                pltpu.make_async_copy(
                    nrows, acc.at[didx.at[i - 2]], nssem).wait()
            if fire_g:
                pltpu.async_copy(feat_hbm.at[sidx.at[i + 2]], nrows, ngsem)

        def stage_body(h, carry0):
            c0 = wid * cpt + h * hcpt
            pltpu.sync_copy(src_hbm.at[pl.ds(c0, hcpt)], sidx)
            pltpu.sync_copy(dst_hbm.at[pl.ds(c0, hcpt)], didx)
            pltpu.sync_copy(w_hbm.at[pl.ds(c0, hcpt)], wv)

            pltpu.async_copy(feat_hbm.at[sidx.at[0]], r0b, gs0)
            pltpu.async_copy(feat_hbm.at[sidx.at[1]], r1b, gs1)
            chunk_step(0, 0, False, True)
            chunk_step(1, 1, False, True)

            def outer_body(io, carry):
                base = 2 + 4 * io
                for lb in range(4):
                    chunk_step(base + lb, (2 + lb) % 4, True, True)
                return carry

            lax.fori_loop(0, (hcpt - 4) // 4, outer_body, 0)
            chunk_step(hcpt - 2, 2, False, False)
            chunk_step(hcpt - 1, 3, False, False)

            for j, (rows, _, ssem) in enumerate(bufs):
                ci = hcpt - 4 + j
                pltpu.make_async_copy(rows, acc.at[didx.at[ci]], ssem).wait()
            return carry0

        lax.fori_loop(0, 4, stage_body, 0)

        plsc.subcore_barrier()
        pltpu.sync_copy(acc.at[pl.ds(r0, rpt)],
                        out_hbm.at[cid, pl.ds(r0, rpt)])

    return run(feat, src2, dst2, w2, zeros)


def _tc_linear(partials, w, b, n):
    d = partials.shape[2]
    o = w.shape[1]
    br = 1000

    def body(p_ref, w_ref, b_ref, o_ref):
        h = p_ref[0] + p_ref[1]
        o_ref[...] = (
            jnp.dot(h, w_ref[...], preferred_element_type=jnp.float32)
            + b_ref[...]
        )

    return pl.pallas_call(
        body,
        grid=(n // br,),
        in_specs=[
            pl.BlockSpec((2, br, d), lambda i: (0, i, 0)),
            pl.BlockSpec((d, o), lambda i: (0, 0)),
            pl.BlockSpec((1, o), lambda i: (0, 0)),
        ],
        out_specs=pl.BlockSpec((br, o), lambda i: (i, 0)),
        out_shape=jax.ShapeDtypeStruct((n, o), jnp.float32),
    )(partials, w, b.reshape(1, o))


def kernel(feat, edge_index, edge_weight, W, b):
    n_nodes, d = feat.shape
    src = edge_index[0].astype(jnp.int32)
    dst = edge_index[1].astype(jnp.int32)
    w = edge_weight.astype(jnp.float32)

    n_edges = src.shape[0]
    group = NW * CHUNK * 8
    ep = -(-n_edges // group) * group
    pad = ep - n_edges
    if pad:
        fill = jnp.arange(pad, dtype=jnp.int32) % n_nodes
        src = jnp.concatenate([src, fill])
        dst = jnp.concatenate([dst, fill])
        w = jnp.pad(w, (0, pad))
    src2 = src.reshape(ep // CHUNK, CHUNK)
    dst2 = dst.reshape(ep // CHUNK, CHUNK)
    w2 = w.reshape(ep // CHUNK, CHUNK)

    np_pad = -(-n_nodes // (NS * 8)) * (NS * 8)
    zeros = jnp.zeros((np_pad, d), jnp.float32)
    partials = _sc_segment_sum(feat, src2, dst2, w2, zeros, np_pad)
    return _tc_linear(partials, W, b, n_nodes)


# speedup vs baseline: 2.6893x; 1.0606x over previous
import functools

import jax
import jax.numpy as jnp
from jax import lax
from jax.experimental import pallas as pl
from jax.experimental.pallas import tpu as pltpu
from jax.experimental.pallas import tpu_sc as plsc

NC = 2
NS = 16
NW = NC * NS
LANES = 16
CHUNK = 128


def _sc_segment_sum(feat, src2, dst2, w2, zeros, n_nodes):
    n_chunks, _ = src2.shape
    d = feat.shape[1]
    cpt = n_chunks // NW
    rpt = n_nodes // NS
    mesh = plsc.VectorSubcoreMesh(core_axis_name="c", subcore_axis_name="s")

    hcpt = cpt // 2

    @functools.partial(
        pl.kernel,
        out_type=jax.ShapeDtypeStruct((NC, n_nodes, d), jnp.float32),
        mesh=mesh,
        scratch_types=[
            pltpu.VMEM((hcpt, CHUNK), jnp.int32),
            pltpu.VMEM((hcpt, CHUNK), jnp.int32),
            pltpu.VMEM((hcpt, CHUNK), jnp.float32),
            pltpu.VMEM((CHUNK, 128), jnp.float32),
            pltpu.VMEM((CHUNK, 128), jnp.float32),
            pltpu.VMEM_SHARED((n_nodes, 128), jnp.float32),
            pltpu.SemaphoreType.DMA,
            pltpu.SemaphoreType.DMA,
        ],
    )
    def run(feat_hbm, src_hbm, dst_hbm, w_hbm, zeros_hbm, out_hbm,
            sidx, didx, wv, rows0, rows1, acc, sem0, sem1):
        cid = lax.axis_index("c")
        tid = lax.axis_index("s")
        wid = cid * NS + tid

        r0 = tid * rpt
        pltpu.sync_copy(zeros_hbm.at[pl.ds(r0, rpt)], acc.at[pl.ds(r0, rpt)])
        plsc.subcore_barrier()

        bufs = ((rows0, sem0), (rows1, sem1))

        def stage_body(h, carry0):
            c0 = wid * cpt + h * hcpt
            pltpu.sync_copy(src_hbm.at[pl.ds(c0, hcpt)], sidx)
            pltpu.sync_copy(dst_hbm.at[pl.ds(c0, hcpt)], didx)
            pltpu.sync_copy(w_hbm.at[pl.ds(c0, hcpt)], wv)

            pltpu.async_copy(feat_hbm.at[sidx.at[0]], rows0, sem0)
            pltpu.async_copy(feat_hbm.at[sidx.at[1]], rows1, sem1)

            def outer_body(io, carry):
                for b, (rows, sem) in enumerate(bufs):
                    i = 2 * io + b
                    pltpu.make_async_copy(
                        feat_hbm.at[sidx.at[i]], rows, sem).wait()

                    def group_body(g, c2, rows=rows, i=i):
                        w16 = wv[i, pl.ds(g * LANES, LANES)]
                        for j in range(LANES):
                            ws = w16[j]
                            e = g * LANES + j
                            for k in range(d // LANES):
                                sl = pl.ds(k * LANES, LANES)
                                rows[e, sl] = rows[e, sl] * ws
                        return c2

                    lax.fori_loop(0, CHUNK // LANES, group_body, 0)
                    pltpu.sync_copy(rows, acc.at[didx.at[i]], add=True)

                    @pl.when(i + 2 < hcpt)
                    def _(rows=rows, sem=sem, i=i):
                        pltpu.async_copy(feat_hbm.at[sidx.at[i + 2]], rows, sem)

                return carry

            lax.fori_loop(0, hcpt // 2, outer_body, 0)
            return carry0

        lax.fori_loop(0, 2, stage_body, 0)

        plsc.subcore_barrier()
        pltpu.sync_copy(acc.at[pl.ds(r0, rpt)],
                        out_hbm.at[cid, pl.ds(r0, rpt)])

    return run(feat, src2, dst2, w2, zeros)


def _tc_linear(partials, w, b, n):
    d = partials.shape[2]
    o = w.shape[1]
    br = 1000

    def body(p_ref, w_ref, b_ref, o_ref):
        h = p_ref[0] + p_ref[1]
        o_ref[...] = (
            jnp.dot(h, w_ref[...], preferred_element_type=jnp.float32)
            + b_ref[...]
        )

    return pl.pallas_call(
        body,
        grid=(n // br,),
        in_specs=[
            pl.BlockSpec((2, br, d), lambda i: (0, i, 0)),
            pl.BlockSpec((d, o), lambda i: (0, 0)),
            pl.BlockSpec((1, o), lambda i: (0, 0)),
        ],
        out_specs=pl.BlockSpec((br, o), lambda i: (i, 0)),
        out_shape=jax.ShapeDtypeStruct((n, o), jnp.float32),
    )(partials, w, b.reshape(1, o))


def kernel(feat, edge_index, edge_weight, W, b):
    n_nodes, d = feat.shape
    src = edge_index[0].astype(jnp.int32)
    dst = edge_index[1].astype(jnp.int32)
    w = edge_weight.astype(jnp.float32)

    n_edges = src.shape[0]
    group = NW * CHUNK * 8
    ep = -(-n_edges // group) * group
    pad = ep - n_edges
    if pad:
        fill = jnp.arange(pad, dtype=jnp.int32) % n_nodes
        src = jnp.concatenate([src, fill])
        dst = jnp.concatenate([dst, fill])
        w = jnp.pad(w, (0, pad))
    src2 = src.reshape(ep // CHUNK, CHUNK)
    dst2 = dst.reshape(ep // CHUNK, CHUNK)
    w2 = w.reshape(ep // CHUNK, CHUNK)

    np_pad = -(-n_nodes // (NS * 8)) * (NS * 8)
    zeros = jnp.zeros((np_pad, d), jnp.float32)
    partials = _sc_segment_sum(feat, src2, dst2, w2, zeros, np_pad)
    return _tc_linear(partials, W, b, n_nodes)
